# Initial kernel scaffold; baseline (speedup 1.0000x reference)
import jax
import jax.numpy as jnp
from jax.experimental import pallas as pl


def kernel(x_bus, x_device, edge_index_bb, edge_index_bd, edge_index_db, edge_attr_bb, edge_attr_bd, edge_attr_db, W_bus, b_bus, W_dev, b_dev, W_edge, b_edge, W1_bb, b1_bb, W1_bd, b1_bd, W1_db, b1_db, W2_bb, b2_bb, W2_bd, b2_bd, W2_db, b2_db, W_rel, b_rel, W_root, W_head, b_head):
    raise NotImplementedError("write your pallas kernel here")


# speedup vs baseline: 2.5124x; 2.5124x over previous
import functools

import jax
import jax.numpy as jnp
import numpy as np
from jax import lax
from jax.experimental import pallas as pl
from jax.experimental.pallas import tpu as pltpu
from jax.experimental.pallas import tpu_sc as plsc

NC = 2
NS = 16
NW = NC * NS
L = 16
HH = 32
CH = 128
NEG = 0.01


def _ru(x, m):
  return (x + m - 1) // m * m


def _lr(x):
  return jnp.maximum(x, 0.0) + NEG * jnp.minimum(x, 0.0)


def _encode_body(x_ref, w_ref, b_ref, o_ref):
  res = _lr(jnp.dot(x_ref[...], w_ref[...],
                    preferred_element_type=jnp.float32) + b_ref[...])
  o_ref[0] = res[:, :HH]
  o_ref[1] = res[:, HH:]


def _encode(x, w, b, n_p, blk):
  n = x.shape[0]
  f = x.shape[1]
  grid = n // blk
  return pl.pallas_call(
      _encode_body,
      grid=(grid,),
      in_specs=[
          pl.BlockSpec((blk, f), lambda i: (i, 0)),
          pl.BlockSpec((f, 2 * HH), lambda i: (0, 0)),
          pl.BlockSpec((1, 2 * HH), lambda i: (0, 0)),
      ],
      out_specs=pl.BlockSpec((2, blk, HH), lambda i: (0, i, 0)),
      out_shape=jax.ShapeDtypeStruct((2, n_p, HH), jnp.float32),
  )(x, w, b.reshape(1, 2 * HH))


def _combine2_body(h_ref, a1_ref, a2_ref, w1_ref, w2_ref, b_ref, o_ref):
  h = jnp.concatenate([h_ref[0], h_ref[1]], axis=1)
  t1 = h + jnp.concatenate([a1_ref[0], a1_ref[1]], axis=1)
  t2 = h + jnp.concatenate([a2_ref[0], a2_ref[1]], axis=1)
  res = jnp.dot(t1, w1_ref[...], preferred_element_type=jnp.float32)
  res = res + jnp.dot(t2, w2_ref[...], preferred_element_type=jnp.float32)
  res = _lr(res + b_ref[...])
  o_ref[0] = res[:, :HH]
  o_ref[1] = res[:, HH:]


def _combine2(h_cat, a1, a2, w1, w2, b, n, n_p, blk):
  grid = n // blk
  spec_h = pl.BlockSpec((2, blk, HH), lambda i: (0, i, 0))
  return pl.pallas_call(
      _combine2_body,
      grid=(grid,),
      in_specs=[
          spec_h, spec_h, spec_h,
          pl.BlockSpec((2 * HH, 2 * HH), lambda i: (0, 0)),
          pl.BlockSpec((2 * HH, 2 * HH), lambda i: (0, 0)),
          pl.BlockSpec((1, 2 * HH), lambda i: (0, 0)),
      ],
      out_specs=pl.BlockSpec((2, blk, HH), lambda i: (0, i, 0)),
      out_shape=jax.ShapeDtypeStruct((2, n_p, HH), jnp.float32),
  )(h_cat, a1, a2, w1, w2, b.reshape(1, 2 * HH))


def _combine1_body(h_ref, a1_ref, w1_ref, b_ref, o_ref):
  h = jnp.concatenate([h_ref[0], h_ref[1]], axis=1)
  t1 = h + jnp.concatenate([a1_ref[0], a1_ref[1]], axis=1)
  res = jnp.dot(t1, w1_ref[...], preferred_element_type=jnp.float32)
  res = _lr(res + b_ref[...])
  o_ref[0] = res[:, :HH]
  o_ref[1] = res[:, HH:]


def _combine1(h_cat, a1, w1, b, n, n_p, blk):
  grid = n // blk
  spec_h = pl.BlockSpec((2, blk, HH), lambda i: (0, i, 0))
  return pl.pallas_call(
      _combine1_body,
      grid=(grid,),
      in_specs=[
          spec_h, spec_h,
          pl.BlockSpec((2 * HH, 2 * HH), lambda i: (0, 0)),
          pl.BlockSpec((1, 2 * HH), lambda i: (0, 0)),
      ],
      out_specs=pl.BlockSpec((2, blk, HH), lambda i: (0, i, 0)),
      out_shape=jax.ShapeDtypeStruct((2, n_p, HH), jnp.float32),
  )(h_cat, a1, w1, b.reshape(1, 2 * HH))


_GDN = lax.GatherDimensionNumbers(
    offset_dims=(), collapsed_slice_dims=(0,), start_index_map=(0,))


def _dyn_gather(vec, idx):
  return lax.gather(vec, idx[:, None], _GDN, slice_sizes=(1,),
                    mode=lax.GatherScatterMode.PROMISE_IN_BOUNDS)


def _bcast(vec, j):
  return _dyn_gather(vec, jnp.full((L,), j, dtype=jnp.int32))


def _make_gine(n_src_p, n_dst, e_pad):
  per_w = e_pad // NS
  n_chunks = per_w // CH
  t_rows = _ru(n_dst, 128) + 128
  n_out = _ru(n_dst, 128)
  zr = t_rows // NS
  orows = n_out // NS

  mesh = plsc.VectorSubcoreMesh(core_axis_name="c", subcore_axis_name="s",
                                num_cores=NC, num_subcores=NS)

  @functools.partial(
      pl.kernel,
      out_type=jax.ShapeDtypeStruct((NC, n_out, HH), jnp.float32),
      mesh=mesh,
      compiler_params=pltpu.CompilerParams(use_tc_tiling_on_sc=False),
      scratch_types=[
          pltpu.VMEM((CH,), jnp.int32),
          pltpu.VMEM((CH,), jnp.int32),
          pltpu.VMEM((CH,), jnp.float32),
          pltpu.VMEM((CH,), jnp.float32),
          pltpu.VMEM((CH, HH), jnp.float32),
          pltpu.VMEM((3, HH), jnp.float32),
          pltpu.VMEM_SHARED((t_rows, HH), jnp.float32),
          pltpu.SemaphoreType.DMA,
      ],
  )
  def gine(h2, src, dst, ea0, ea1, wpack, zrs, out,
           src_v, dst_v, ea0_v, ea1_v, rows_v, w_v, acc, sem):
    c = lax.axis_index("c")
    s = lax.axis_index("s")
    base = s * per_w
    coff = c * n_src_p

    pltpu.sync_copy(wpack.at[c], w_v)
    pltpu.sync_copy(zrs, acc.at[pl.ds(s * zr, zr)])
    plsc.subcore_barrier()

    we0 = [w_v[0, pl.ds(0, L)], w_v[0, pl.ds(L, L)]]
    we1 = [w_v[1, pl.ds(0, L)], w_v[1, pl.ds(L, L)]]
    be = [w_v[2, pl.ds(0, L)], w_v[2, pl.ds(L, L)]]

    def chunk(g, carry):
      off = base + g * CH
      pltpu.sync_copy(src.at[pl.ds(off, CH)], src_v)
      pltpu.sync_copy(dst.at[pl.ds(off, CH)], dst_v)
      pltpu.sync_copy(ea0.at[pl.ds(off, CH)], ea0_v)
      pltpu.sync_copy(ea1.at[pl.ds(off, CH)], ea1_v)
      for t in range(CH // L):
        sl = pl.ds(t * L, L)
        src_v[sl] = src_v[sl] + coff
      pltpu.async_copy(h2.at[src_v], rows_v, sem).wait()
      for g16 in range(CH // L):
        va0 = ea0_v[pl.ds(g16 * L, L)]
        va1 = ea1_v[pl.ds(g16 * L, L)]
        for j in range(L):
          a0 = _bcast(va0, j)
          a1 = _bcast(va1, j)
          r = g16 * L + j
          for t in range(2):
            sl = pl.ds(t * L, L)
            e = we0[t] * a0 + we1[t] * a1 + be[t]
            e = jnp.maximum(e, 0.0) + NEG * jnp.minimum(e, 0.0)
            rows_v[r, sl] = jnp.maximum(rows_v[r, sl] + e, 0.0)
      pltpu.sync_copy(rows_v, acc.at[dst_v], add=True)
      return carry

    lax.fori_loop(0, n_chunks, chunk, 0)
    plsc.subcore_barrier()
    pltpu.sync_copy(acc.at[pl.ds(s * orows, orows)],
                    out.at[c, pl.ds(s * orows, orows)])

  return gine


def _make_pool(n_src_p, n_out, e_pad):
  per_w = e_pad // NS
  n_chunks = per_w // CH
  t_rows = n_out + 128
  zr = t_rows // NS
  orows = n_out // NS

  mesh = plsc.VectorSubcoreMesh(core_axis_name="c", subcore_axis_name="s",
                                num_cores=NC, num_subcores=NS)

  @functools.partial(
      pl.kernel,
      out_type=jax.ShapeDtypeStruct((NC, n_out, HH), jnp.float32),
      mesh=mesh,
      compiler_params=pltpu.CompilerParams(use_tc_tiling_on_sc=False),
      scratch_types=[
          pltpu.VMEM((CH,), jnp.int32),
          pltpu.VMEM((CH,), jnp.int32),
          pltpu.VMEM((CH, HH), jnp.float32),
          pltpu.VMEM_SHARED((t_rows, HH), jnp.float32),
          pltpu.SemaphoreType.DMA,
      ],
  )
  def pool(h2, src, dst, zrs, out, src_v, dst_v, rows_v, acc, sem):
    c = lax.axis_index("c")
    s = lax.axis_index("s")
    base = s * per_w
    coff = c * n_src_p

    pltpu.sync_copy(zrs, acc.at[pl.ds(s * zr, zr)])
    plsc.subcore_barrier()

    def chunk(g, carry):
      off = base + g * CH
      pltpu.sync_copy(src.at[pl.ds(off, CH)], src_v)
      pltpu.sync_copy(dst.at[pl.ds(off, CH)], dst_v)
      for t in range(CH // L):
        sl = pl.ds(t * L, L)
        src_v[sl] = src_v[sl] + coff
      pltpu.sync_copy(h2.at[src_v], rows_v)
      pltpu.sync_copy(rows_v, acc.at[dst_v], add=True)
      return carry

    lax.fori_loop(0, n_chunks, chunk, 0)
    plsc.subcore_barrier()
    pltpu.sync_copy(acc.at[pl.ds(s * orows, orows)],
                    out.at[c, pl.ds(s * orows, orows)])

  return pool


def _make_final(nb, nb_p, stripe, k):
  spw = stripe
  cdims = (((1,), (1,)), ((), ()))

  def body(h_ref, a_ref, wrw_ref, wh_ref, brel_ref, bhead_ref, o_ref,
           sc_s, w_s, tot_s):
    swp = pl.program_id(0)
    t = pl.program_id(1)
    rh = jnp.concatenate([h_ref[0, 0], h_ref[1, 0]], axis=1)

    @pl.when(swp == 0)
    def _sweep0():
      ra = jnp.concatenate([a_ref[0, 0], a_ref[1, 0]], axis=1)
      sr = lax.dot_general(wrw_ref[0:1], ra, cdims,
                           preferred_element_type=jnp.float32)
      sr = sr + lax.dot_general(wrw_ref[1:2], rh, cdims,
                                preferred_element_type=jnp.float32)
      sc_s[pl.ds(t, 1)] = jnp.tanh(sr + brel_ref[0, 0])

    @pl.when((swp == 1) & (t == 0))
    def _thresh():
      imin = jnp.int32(-2**31)
      score = sc_s[...]
      jcol = lax.broadcasted_iota(jnp.int32, (NS, spw), 1)
      trow = lax.broadcasted_iota(jnp.int32, (NS, spw), 0)
      node = trow * stripe + jcol
      valid = node < nb
      u = lax.bitcast_convert_type(score, jnp.int32)
      skey = u ^ jnp.where(u < 0, jnp.int32(0x7FFFFFFF), jnp.int32(0))
      skey = jnp.where(valid, skey, imin)

      def sbit(b, p):
        cand = p | (jnp.int32(1) << b)
        thr = imin + cand
        cnt = jnp.sum((skey >= thr).astype(jnp.int32))
        return jnp.where(cnt >= k, cand, p)

      p = lax.fori_loop(0, 32, lambda i, p: sbit(31 - i, p), jnp.int32(0))
      tkey = imin + p
      gt = skey > tkey
      ties = skey == tkey
      r = k - jnp.sum(gt.astype(jnp.int32))
      m = jnp.int32(65535) - node

      def ibit(b, p2):
        cand = p2 | (jnp.int32(1) << b)
        cnt = jnp.sum((ties & (m >= cand)).astype(jnp.int32))
        return jnp.where(cnt >= r, cand, p2)

      p2 = lax.fori_loop(0, 16, lambda i, p2: ibit(15 - i, p2), jnp.int32(0))
      tie_in = ties & (m >= p2) & (r > 0)
      w_s[...] = jnp.where(gt | tie_in, score, 0.0)
      tot_s[...] = jnp.zeros((8, 2 * HH), jnp.float32)

    @pl.when(swp == 1)
    def _sweep1():
      rmask2 = (t * stripe
                + lax.broadcasted_iota(jnp.int32, (1, spw), 1)) < nb
      lhs = jnp.concatenate([rmask2.astype(jnp.float32), w_s[pl.ds(t, 1)]],
                            axis=0)
      rmask = (t * stripe
               + lax.broadcasted_iota(jnp.int32, (spw, 2 * HH), 0)) < nb
      hz = jnp.where(rmask, rh, 0.0)
      tot_s[0:2] = tot_s[0:2] + jnp.dot(lhs, hz,
                                        preferred_element_type=jnp.float32)

    @pl.when((swp == 1) & (t == NS - 1))
    def _finish():
      gf = tot_s[0:1] / np.float32(nb)
      lf = tot_s[1:2] / np.float32(k)
      res = (jnp.sum(gf * wh_ref[0:1]) + jnp.sum(lf * wh_ref[1:2])
             + bhead_ref[0, 0])
      o_ref[...] = res.reshape(1, 1)

  def run(h_cat, aggp, wrw, w_head, b_rel, b_head):
    h4 = h_cat.reshape(2, NS, stripe, HH)
    a4 = aggp.reshape(2, NS, stripe, HH)
    blk = pl.BlockSpec((2, 1, stripe, HH), lambda s, t: (0, t, 0, 0))
    return pl.pallas_call(
        body,
        grid=(2, NS),
        in_specs=[
            blk, blk,
            pl.BlockSpec((2, 2 * HH), lambda s, t: (0, 0)),
            pl.BlockSpec((2, 2 * HH), lambda s, t: (0, 0)),
            pl.BlockSpec((1, 1), lambda s, t: (0, 0)),
            pl.BlockSpec((1, 1), lambda s, t: (0, 0)),
        ],
        out_specs=pl.BlockSpec((1, 1), lambda s, t: (0, 0)),
        out_shape=jax.ShapeDtypeStruct((1, 1), jnp.float32),
        scratch_shapes=[
            pltpu.VMEM((NS, spw), jnp.float32),
            pltpu.VMEM((NS, spw), jnp.float32),
            pltpu.VMEM((8, 2 * HH), jnp.float32),
        ],
    )(h4, a4, wrw, w_head.reshape(2, 2 * HH), b_rel.reshape(1, 1),
      b_head.reshape(1, 1))

  return run


def kernel(x_bus, x_device, edge_index_bb, edge_index_bd, edge_index_db,
           edge_attr_bb, edge_attr_bd, edge_attr_db, W_bus, b_bus, W_dev,
           b_dev, W_edge, b_edge, W1_bb, b1_bb, W1_bd, b1_bd, W1_db, b1_db,
           W2_bb, b2_bb, W2_bd, b2_bd, W2_db, b2_db, W_rel, b_rel, W_root,
           W_head, b_head):
  nb = x_bus.shape[0]
  nd = x_device.shape[0]
  e = edge_index_bb.shape[1]
  stripe = _ru(-(-nb // NS), 448)
  nb_p = stripe * NS
  nd_p = _ru(nd, 128)
  e_pad = _ru(e, NW * CH)
  pads = e_pad - e

  def pad_edges(ei, n_src, n_dst):
    ar = jnp.arange(pads, dtype=jnp.int32)
    src = jnp.concatenate([ei[0], ar % n_src])
    dst = jnp.concatenate([ei[1], n_dst + (ar % 128)])
    return src, dst

  src_bb, dst_bb = pad_edges(edge_index_bb, nb, nb)
  src_bd, dst_bd = pad_edges(edge_index_bd, nb, nd)
  src_db, dst_db = pad_edges(edge_index_db, nd, nb)
  zpad = jnp.zeros((pads,), jnp.float32)
  ea0_bb = jnp.concatenate([edge_attr_bb[:, 0], zpad])
  ea1_bb = jnp.concatenate([edge_attr_bb[:, 1], zpad])
  ea0_bd = jnp.concatenate([edge_attr_bd[:, 0], zpad])
  ea1_bd = jnp.concatenate([edge_attr_bd[:, 1], zpad])
  ea0_db = jnp.concatenate([edge_attr_db[:, 0], zpad])
  ea1_db = jnp.concatenate([edge_attr_db[:, 1], zpad])

  wpack_e = jnp.stack([W_edge[0].reshape(2, HH), W_edge[1].reshape(2, HH),
                       b_edge.reshape(2, HH)], axis=1)
  wrw = jnp.stack([W_rel[:, 0], W_root[:, 0]], axis=0)

  t_rows_b = _ru(nb, 128) + 128
  t_rows_d = _ru(nd, 128) + 128
  zrs_b = jnp.zeros((t_rows_b // NS, HH), jnp.float32)
  zrs_d = jnp.zeros((t_rows_d // NS, HH), jnp.float32)

  hb = _encode(x_bus, W_bus, b_bus, nb_p, 2000)
  hd = _encode(x_device, W_dev, b_dev, nd_p, 1000)

  gine_bb = _make_gine(nb_p, nb, e_pad)
  gine_bd = _make_gine(nb_p, nd, e_pad)
  gine_db = _make_gine(nd_p, nb, e_pad)

  def layer(hb_c, hd_c, w_bb, b_bb, w_bd, b_bd, w_db, b_db):
    hb2 = hb_c.reshape(2 * nb_p, HH)
    hd2 = hd_c.reshape(2 * nd_p, HH)
    a_bb = gine_bb(hb2, src_bb, dst_bb, ea0_bb, ea1_bb, wpack_e, zrs_b)
    a_db = gine_db(hd2, src_db, dst_db, ea0_db, ea1_db, wpack_e, zrs_b)
    a_bd = gine_bd(hb2, src_bd, dst_bd, ea0_bd, ea1_bd, wpack_e, zrs_d)
    hb_n = _combine2(hb_c, a_bb, a_db, w_bb, w_db, b_bb + b_db,
                     nb, nb_p, 2000)
    hd_n = _combine1(hd_c, a_bd, w_bd, b_bd, nd, nd_p, 1000)
    return hb_n, hd_n

  hb, hd = layer(hb, hd, W1_bb, b1_bb, W1_bd, b1_bd, W1_db, b1_db)
  hb, hd = layer(hb, hd, W2_bb, b2_bb, W2_bd, b2_bd, W2_db, b2_db)

  zrs_p = jnp.zeros(((nb_p + 128) // NS, HH), jnp.float32)
  pool = _make_pool(nb_p, nb_p, e_pad)
  aggp = pool(hb.reshape(2 * nb_p, HH), src_bb, dst_bb, zrs_p)

  k = int(np.ceil(0.5 * nb))
  final = _make_final(nb, nb_p, stripe, k)
  return final(hb, aggp, wrw, W_head, b_rel, b_head)


# speedup vs baseline: 3.8864x; 1.5469x over previous
import functools

import jax
import jax.numpy as jnp
import numpy as np
from jax import lax
from jax.experimental import pallas as pl
from jax.experimental.pallas import tpu as pltpu
from jax.experimental.pallas import tpu_sc as plsc

NC = 2
NS = 16
NW = NC * NS
L = 16
HH = 32
CH = 128
NEG = 0.01


def _ru(x, m):
  return (x + m - 1) // m * m


def _lr(x):
  return jnp.maximum(x, 0.0) + NEG * jnp.minimum(x, 0.0)


def _encode_body(x_ref, w_ref, b_ref, o_ref):
  res = _lr(jnp.dot(x_ref[...], w_ref[...],
                    preferred_element_type=jnp.float32) + b_ref[...])
  o_ref[0] = res[:, :HH]
  o_ref[1] = res[:, HH:]


def _encode(x, w, b, n_p, blk):
  n = x.shape[0]
  f = x.shape[1]
  grid = n // blk
  return pl.pallas_call(
      _encode_body,
      grid=(grid,),
      in_specs=[
          pl.BlockSpec((blk, f), lambda i: (i, 0)),
          pl.BlockSpec((f, 2 * HH), lambda i: (0, 0)),
          pl.BlockSpec((1, 2 * HH), lambda i: (0, 0)),
      ],
      out_specs=pl.BlockSpec((2, blk, HH), lambda i: (0, i, 0)),
      out_shape=jax.ShapeDtypeStruct((2, n_p, HH), jnp.float32),
  )(x, w, b.reshape(1, 2 * HH))


def _combine2_body(h_ref, a1_ref, a2_ref, w1_ref, w2_ref, b_ref, o_ref):
  h = jnp.concatenate([h_ref[0], h_ref[1]], axis=1)
  t1 = h + jnp.concatenate([a1_ref[0], a1_ref[1]], axis=1)
  t2 = h + jnp.concatenate([a2_ref[0], a2_ref[1]], axis=1)
  res = jnp.dot(t1, w1_ref[...], preferred_element_type=jnp.float32)
  res = res + jnp.dot(t2, w2_ref[...], preferred_element_type=jnp.float32)
  res = _lr(res + b_ref[...])
  o_ref[0] = res[:, :HH]
  o_ref[1] = res[:, HH:]


def _combine2(h_cat, a1, a2, w1, w2, b, n, n_p, blk):
  grid = n // blk
  spec_h = pl.BlockSpec((2, blk, HH), lambda i: (0, i, 0))
  return pl.pallas_call(
      _combine2_body,
      grid=(grid,),
      in_specs=[
          spec_h, spec_h, spec_h,
          pl.BlockSpec((2 * HH, 2 * HH), lambda i: (0, 0)),
          pl.BlockSpec((2 * HH, 2 * HH), lambda i: (0, 0)),
          pl.BlockSpec((1, 2 * HH), lambda i: (0, 0)),
      ],
      out_specs=pl.BlockSpec((2, blk, HH), lambda i: (0, i, 0)),
      out_shape=jax.ShapeDtypeStruct((2, n_p, HH), jnp.float32),
  )(h_cat, a1, a2, w1, w2, b.reshape(1, 2 * HH))


def _combine1_body(h_ref, a1_ref, w1_ref, b_ref, o_ref):
  h = jnp.concatenate([h_ref[0], h_ref[1]], axis=1)
  t1 = h + jnp.concatenate([a1_ref[0], a1_ref[1]], axis=1)
  res = jnp.dot(t1, w1_ref[...], preferred_element_type=jnp.float32)
  res = _lr(res + b_ref[...])
  o_ref[0] = res[:, :HH]
  o_ref[1] = res[:, HH:]


def _combine1(h_cat, a1, w1, b, n, n_p, blk):
  grid = n // blk
  spec_h = pl.BlockSpec((2, blk, HH), lambda i: (0, i, 0))
  return pl.pallas_call(
      _combine1_body,
      grid=(grid,),
      in_specs=[
          spec_h, spec_h,
          pl.BlockSpec((2 * HH, 2 * HH), lambda i: (0, 0)),
          pl.BlockSpec((1, 2 * HH), lambda i: (0, 0)),
      ],
      out_specs=pl.BlockSpec((2, blk, HH), lambda i: (0, i, 0)),
      out_shape=jax.ShapeDtypeStruct((2, n_p, HH), jnp.float32),
  )(h_cat, a1, w1, b.reshape(1, 2 * HH))


_GDN = lax.GatherDimensionNumbers(
    offset_dims=(), collapsed_slice_dims=(0,), start_index_map=(0,))


def _dyn_gather(vec, idx):
  return lax.gather(vec, idx[:, None], _GDN, slice_sizes=(1,),
                    mode=lax.GatherScatterMode.PROMISE_IN_BOUNDS)


def _bcast(vec, j):
  return _dyn_gather(vec, jnp.full((L,), j, dtype=jnp.int32))


SB = 1024
SR = SB // CH


def _make_gine(n_src_p, n_dst, e_pad):
  per_w = e_pad // NS
  nblk = per_w // SB
  t_rows = _ru(n_dst, 128) + 128
  n_out = _ru(n_dst, 128)
  zr = t_rows // NS
  orows = n_out // NS

  mesh = plsc.VectorSubcoreMesh(core_axis_name="c", subcore_axis_name="s",
                                num_cores=NC, num_subcores=NS)

  @functools.partial(
      pl.kernel,
      out_type=jax.ShapeDtypeStruct((NC, n_out, HH), jnp.float32),
      mesh=mesh,
      compiler_params=pltpu.CompilerParams(use_tc_tiling_on_sc=False),
      scratch_types=[
          pltpu.VMEM((SR, CH), jnp.int32),
          pltpu.VMEM((SR, CH), jnp.int32),
          pltpu.VMEM((SR, CH), jnp.float32),
          pltpu.VMEM((SR, CH), jnp.float32),
          pltpu.VMEM((CH, HH), jnp.float32),
          pltpu.VMEM((CH, HH), jnp.float32),
          pltpu.VMEM((3, HH), jnp.float32),
          pltpu.VMEM_SHARED((t_rows, HH), jnp.float32),
          pltpu.SemaphoreType.DMA,
          pltpu.SemaphoreType.DMA,
      ],
  )
  def gine(h2, src2, dst2, ea02, ea12, wpack, zrs, out,
           src_v, dst_v, ea0_v, ea1_v, rows_a, rows_b, w_v, acc,
           sem_a, sem_b):
    c = lax.axis_index("c")
    s = lax.axis_index("s")
    rbase = s * (per_w // CH)
    coff = c * n_src_p

    pltpu.sync_copy(wpack.at[c], w_v)
    pltpu.sync_copy(zrs, acc.at[pl.ds(s * zr, zr)])
    plsc.subcore_barrier()

    we0 = [w_v[0, pl.ds(0, L)], w_v[0, pl.ds(L, L)]]
    we1 = [w_v[1, pl.ds(0, L)], w_v[1, pl.ds(L, L)]]
    be = [w_v[2, pl.ds(0, L)], w_v[2, pl.ds(L, L)]]

    def compute(jj, rows):
      for g16 in range(CH // L):
        va0 = ea0_v[jj, pl.ds(g16 * L, L)]
        va1 = ea1_v[jj, pl.ds(g16 * L, L)]
        for j in range(L):
          a0 = _bcast(va0, j)
          a1 = _bcast(va1, j)
          r = g16 * L + j
          for t in range(2):
            sl = pl.ds(t * L, L)
            e = we0[t] * a0 + we1[t] * a1 + be[t]
            e = jnp.maximum(e, 0.0) + NEG * jnp.minimum(e, 0.0)
            rows[r, sl] = jnp.maximum(rows[r, sl] + e, 0.0)

    def block(b, carry):
      row0 = rbase + b * SR
      pltpu.sync_copy(src2.at[pl.ds(row0, SR)], src_v)
      pltpu.sync_copy(dst2.at[pl.ds(row0, SR)], dst_v)
      pltpu.sync_copy(ea02.at[pl.ds(row0, SR)], ea0_v)
      pltpu.sync_copy(ea12.at[pl.ds(row0, SR)], ea1_v)
      for r in range(SR):
        for g in range(CH // L):
          sl = pl.ds(g * L, L)
          src_v[r, sl] = src_v[r, sl] + coff
      pltpu.async_copy(h2.at[src_v.at[0]], rows_a, sem_a)

      def inner(jp, cin):
        j0 = jp * 2
        pltpu.make_async_copy(h2.at[src_v.at[j0]], rows_a, sem_a).wait()
        pltpu.async_copy(h2.at[src_v.at[j0 + 1]], rows_b, sem_b)
        compute(j0, rows_a)
        pltpu.sync_copy(rows_a, acc.at[dst_v.at[j0]], add=True)
        pltpu.make_async_copy(h2.at[src_v.at[j0 + 1]], rows_b, sem_b).wait()

        @pl.when(jp < SR // 2 - 1)
        def _():
          pltpu.async_copy(h2.at[src_v.at[j0 + 2]], rows_a, sem_a)

        compute(j0 + 1, rows_b)
        pltpu.sync_copy(rows_b, acc.at[dst_v.at[j0 + 1]], add=True)
        return cin

      lax.fori_loop(0, SR // 2, inner, 0)
      return carry

    lax.fori_loop(0, nblk, block, 0)
    plsc.subcore_barrier()
    pltpu.sync_copy(acc.at[pl.ds(s * orows, orows)],
                    out.at[c, pl.ds(s * orows, orows)])

  return gine


def _make_pool(n_src_p, n_out, e_pad):
  per_w = e_pad // NS
  nblk = per_w // SB
  t_rows = n_out + 128
  zr = t_rows // NS
  orows = n_out // NS

  mesh = plsc.VectorSubcoreMesh(core_axis_name="c", subcore_axis_name="s",
                                num_cores=NC, num_subcores=NS)

  @functools.partial(
      pl.kernel,
      out_type=jax.ShapeDtypeStruct((NC, n_out, HH), jnp.float32),
      mesh=mesh,
      compiler_params=pltpu.CompilerParams(use_tc_tiling_on_sc=False),
      scratch_types=[
          pltpu.VMEM((SR, CH), jnp.int32),
          pltpu.VMEM((SR, CH), jnp.int32),
          pltpu.VMEM((CH, HH), jnp.float32),
          pltpu.VMEM((CH, HH), jnp.float32),
          pltpu.VMEM_SHARED((t_rows, HH), jnp.float32),
          pltpu.SemaphoreType.DMA,
          pltpu.SemaphoreType.DMA,
      ],
  )
  def pool(h2, src2, dst2, zrs, out, src_v, dst_v, rows_a, rows_b, acc,
           sem_a, sem_b):
    c = lax.axis_index("c")
    s = lax.axis_index("s")
    rbase = s * (per_w // CH)
    coff = c * n_src_p

    pltpu.sync_copy(zrs, acc.at[pl.ds(s * zr, zr)])
    plsc.subcore_barrier()

    def block(b, carry):
      row0 = rbase + b * SR
      pltpu.sync_copy(src2.at[pl.ds(row0, SR)], src_v)
      pltpu.sync_copy(dst2.at[pl.ds(row0, SR)], dst_v)
      for r in range(SR):
        for g in range(CH // L):
          sl = pl.ds(g * L, L)
          src_v[r, sl] = src_v[r, sl] + coff
      pltpu.async_copy(h2.at[src_v.at[0]], rows_a, sem_a)

      def inner(jp, cin):
        j0 = jp * 2
        pltpu.make_async_copy(h2.at[src_v.at[j0]], rows_a, sem_a).wait()
        pltpu.async_copy(h2.at[src_v.at[j0 + 1]], rows_b, sem_b)
        pltpu.sync_copy(rows_a, acc.at[dst_v.at[j0]], add=True)
        pltpu.make_async_copy(h2.at[src_v.at[j0 + 1]], rows_b, sem_b).wait()

        @pl.when(jp < SR // 2 - 1)
        def _():
          pltpu.async_copy(h2.at[src_v.at[j0 + 2]], rows_a, sem_a)

        pltpu.sync_copy(rows_b, acc.at[dst_v.at[j0 + 1]], add=True)
        return cin

      lax.fori_loop(0, SR // 2, inner, 0)
      return carry

    lax.fori_loop(0, nblk, block, 0)
    plsc.subcore_barrier()
    pltpu.sync_copy(acc.at[pl.ds(s * orows, orows)],
                    out.at[c, pl.ds(s * orows, orows)])

  return pool


def _make_final(nb, nb_p, stripe, k):
  spw = stripe
  cdims = (((1,), (1,)), ((), ()))

  def body(h_ref, a_ref, wrw_ref, wh_ref, brel_ref, bhead_ref, o_ref,
           sc_s, w_s, tot_s):
    swp = pl.program_id(0)
    t = pl.program_id(1)
    rh = jnp.concatenate([h_ref[0, 0], h_ref[1, 0]], axis=1)

    @pl.when(swp == 0)
    def _sweep0():
      ra = jnp.concatenate([a_ref[0, 0], a_ref[1, 0]], axis=1)
      sr = lax.dot_general(wrw_ref[0:1], ra, cdims,
                           preferred_element_type=jnp.float32)
      sr = sr + lax.dot_general(wrw_ref[1:2], rh, cdims,
                                preferred_element_type=jnp.float32)
      sc_s[pl.ds(t, 1)] = jnp.tanh(sr + brel_ref[0, 0])

    @pl.when((swp == 1) & (t == 0))
    def _thresh():
      imin = jnp.int32(-2**31)
      score = sc_s[...]
      jcol = lax.broadcasted_iota(jnp.int32, (NS, spw), 1)
      trow = lax.broadcasted_iota(jnp.int32, (NS, spw), 0)
      node = trow * stripe + jcol
      valid = node < nb
      u = lax.bitcast_convert_type(score, jnp.int32)
      skey = u ^ jnp.where(u < 0, jnp.int32(0x7FFFFFFF), jnp.int32(0))
      skey = jnp.where(valid, skey, imin)

      def sbit(b, p):
        cand = p | (jnp.int32(1) << b)
        thr = imin + cand
        cnt = jnp.sum((skey >= thr).astype(jnp.int32))
        return jnp.where(cnt >= k, cand, p)

      p = lax.fori_loop(0, 32, lambda i, p: sbit(31 - i, p), jnp.int32(0))
      tkey = imin + p
      gt = skey > tkey
      ties = skey == tkey
      r = k - jnp.sum(gt.astype(jnp.int32))
      m = jnp.int32(65535) - node

      def ibit(b, p2):
        cand = p2 | (jnp.int32(1) << b)
        cnt = jnp.sum((ties & (m >= cand)).astype(jnp.int32))
        return jnp.where(cnt >= r, cand, p2)

      p2 = lax.fori_loop(0, 16, lambda i, p2: ibit(15 - i, p2), jnp.int32(0))
      tie_in = ties & (m >= p2) & (r > 0)
      w_s[...] = jnp.where(gt | tie_in, score, 0.0)
      tot_s[...] = jnp.zeros((8, 2 * HH), jnp.float32)

    @pl.when(swp == 1)
    def _sweep1():
      rmask2 = (t * stripe
                + lax.broadcasted_iota(jnp.int32, (1, spw), 1)) < nb
      lhs = jnp.concatenate([rmask2.astype(jnp.float32), w_s[pl.ds(t, 1)]],
                            axis=0)
      rmask = (t * stripe
               + lax.broadcasted_iota(jnp.int32, (spw, 2 * HH), 0)) < nb
      hz = jnp.where(rmask, rh, 0.0)
      tot_s[0:2] = tot_s[0:2] + jnp.dot(lhs, hz,
                                        preferred_element_type=jnp.float32)

    @pl.when((swp == 1) & (t == NS - 1))
    def _finish():
      gf = tot_s[0:1] / np.float32(nb)
      lf = tot_s[1:2] / np.float32(k)
      res = (jnp.sum(gf * wh_ref[0:1]) + jnp.sum(lf * wh_ref[1:2])
             + bhead_ref[0, 0])
      o_ref[...] = res.reshape(1, 1)

  def run(h_cat, aggp, wrw, w_head, b_rel, b_head):
    h4 = h_cat.reshape(2, NS, stripe, HH)
    a4 = aggp.reshape(2, NS, stripe, HH)
    blk = pl.BlockSpec((2, 1, stripe, HH), lambda s, t: (0, t, 0, 0))
    return pl.pallas_call(
        body,
        grid=(2, NS),
        in_specs=[
            blk, blk,
            pl.BlockSpec((2, 2 * HH), lambda s, t: (0, 0)),
            pl.BlockSpec((2, 2 * HH), lambda s, t: (0, 0)),
            pl.BlockSpec((1, 1), lambda s, t: (0, 0)),
            pl.BlockSpec((1, 1), lambda s, t: (0, 0)),
        ],
        out_specs=pl.BlockSpec((1, 1), lambda s, t: (0, 0)),
        out_shape=jax.ShapeDtypeStruct((1, 1), jnp.float32),
        scratch_shapes=[
            pltpu.VMEM((NS, spw), jnp.float32),
            pltpu.VMEM((NS, spw), jnp.float32),
            pltpu.VMEM((8, 2 * HH), jnp.float32),
        ],
    )(h4, a4, wrw, w_head.reshape(2, 2 * HH), b_rel.reshape(1, 1),
      b_head.reshape(1, 1))

  return run


def kernel(x_bus, x_device, edge_index_bb, edge_index_bd, edge_index_db,
           edge_attr_bb, edge_attr_bd, edge_attr_db, W_bus, b_bus, W_dev,
           b_dev, W_edge, b_edge, W1_bb, b1_bb, W1_bd, b1_bd, W1_db, b1_db,
           W2_bb, b2_bb, W2_bd, b2_bd, W2_db, b2_db, W_rel, b_rel, W_root,
           W_head, b_head):
  nb = x_bus.shape[0]
  nd = x_device.shape[0]
  e = edge_index_bb.shape[1]
  stripe = _ru(-(-nb // NS), 448)
  nb_p = stripe * NS
  nd_p = _ru(nd, 128)
  e_pad = _ru(e, NW * CH)
  pads = e_pad - e

  def pad_edges(ei, n_src, n_dst):
    ar = jnp.arange(pads, dtype=jnp.int32)
    src = jnp.concatenate([ei[0], ar % n_src]).reshape(e_pad // CH, CH)
    dst = jnp.concatenate([ei[1], n_dst + (ar % 128)]).reshape(
        e_pad // CH, CH)
    return src, dst

  def pad2(v):
    return jnp.concatenate(
        [v, jnp.zeros((pads,), jnp.float32)]).reshape(e_pad // CH, CH)

  src_bb, dst_bb = pad_edges(edge_index_bb, nb, nb)
  src_bd, dst_bd = pad_edges(edge_index_bd, nb, nd)
  src_db, dst_db = pad_edges(edge_index_db, nd, nb)
  ea0_bb = pad2(edge_attr_bb[:, 0])
  ea1_bb = pad2(edge_attr_bb[:, 1])
  ea0_bd = pad2(edge_attr_bd[:, 0])
  ea1_bd = pad2(edge_attr_bd[:, 1])
  ea0_db = pad2(edge_attr_db[:, 0])
  ea1_db = pad2(edge_attr_db[:, 1])

  wpack_e = jnp.stack([W_edge[0].reshape(2, HH), W_edge[1].reshape(2, HH),
                       b_edge.reshape(2, HH)], axis=1)
  wrw = jnp.stack([W_rel[:, 0], W_root[:, 0]], axis=0)

  t_rows_b = _ru(nb, 128) + 128
  t_rows_d = _ru(nd, 128) + 128
  zrs_b = jnp.zeros((t_rows_b // NS, HH), jnp.float32)
  zrs_d = jnp.zeros((t_rows_d // NS, HH), jnp.float32)

  hb = _encode(x_bus, W_bus, b_bus, nb_p, 2000)
  hd = _encode(x_device, W_dev, b_dev, nd_p, 1000)

  gine_bb = _make_gine(nb_p, nb, e_pad)
  gine_bd = _make_gine(nb_p, nd, e_pad)
  gine_db = _make_gine(nd_p, nb, e_pad)

  def layer(hb_c, hd_c, w_bb, b_bb, w_bd, b_bd, w_db, b_db):
    hb2 = hb_c.reshape(2 * nb_p, HH)
    hd2 = hd_c.reshape(2 * nd_p, HH)
    a_bb = gine_bb(hb2, src_bb, dst_bb, ea0_bb, ea1_bb, wpack_e, zrs_b)
    a_db = gine_db(hd2, src_db, dst_db, ea0_db, ea1_db, wpack_e, zrs_b)
    a_bd = gine_bd(hb2, src_bd, dst_bd, ea0_bd, ea1_bd, wpack_e, zrs_d)
    hb_n = _combine2(hb_c, a_bb, a_db, w_bb, w_db, b_bb + b_db,
                     nb, nb_p, 2000)
    hd_n = _combine1(hd_c, a_bd, w_bd, b_bd, nd, nd_p, 1000)
    return hb_n, hd_n

  hb, hd = layer(hb, hd, W1_bb, b1_bb, W1_bd, b1_bd, W1_db, b1_db)
  hb, hd = layer(hb, hd, W2_bb, b2_bb, W2_bd, b2_bd, W2_db, b2_db)

  zrs_p = jnp.zeros(((nb_p + 128) // NS, HH), jnp.float32)
  pool = _make_pool(nb_p, nb_p, e_pad)
  aggp = pool(hb.reshape(2 * nb_p, HH), src_bb, dst_bb, zrs_p)

  k = int(np.ceil(0.5 * nb))
  final = _make_final(nb, nb_p, stripe, k)
  return final(hb, aggp, wrw, W_head, b_rel, b_head)
